# Initial kernel scaffold; baseline (speedup 1.0000x reference)
import jax
import jax.numpy as jnp
from jax.experimental import pallas as pl


def kernel(x_s, x_t, edge_index, edge_attr, W1, b1, a1, W2, b2, F):
    raise NotImplementedError("write your pallas kernel here")


# speedup vs baseline: 1.4097x; 1.4097x over previous
import jax
import jax.numpy as jnp
from jax import lax
from jax.experimental import pallas as pl
from jax.experimental.pallas import tpu as pltpu
from jax.experimental.pallas import tpu_sc as plsc

_N = 10000
_E = 320000
_DS = 128
_DT = 128
_DE = 16
_K = 8
_HID = 2 * _DT
_EIN = _DE + 2 * _DS + 2 * _DT
_TBLW = _DS + _DT

_NC = 2
_NS = 16
_NW = _NC * _NS
_PER_W = _E // _NW
_CH = 80
_NCHUNK = _PER_W // _CH
_NPAD = 10240
_NODES_W = _NPAD // _NW
_SCAP = 12800
_SCH = _SCAP // _CH
_DUMP = _NPAD - 1


def _sc_mesh():
    return plsc.VectorSubcoreMesh(core_axis_name="c", subcore_axis_name="s")


def _sc_gather(u, fidx, tidx):
    def body(u_h, fi_h, ti_h, rf_h, rt_h,
             fi_v, ti_v, bf_v, bt_v, s1, s2):
        wid = lax.axis_index("s") * _NC + lax.axis_index("c")
        base = wid * _PER_W

        def chunk(i, carry):
            off = base + i * _CH
            pltpu.sync_copy(fi_h.at[pl.ds(off, _CH)], fi_v)
            pltpu.sync_copy(ti_h.at[pl.ds(off, _CH)], ti_v)
            cf = pltpu.async_copy(u_h.at[fi_v], bf_v, s1)
            ct = pltpu.async_copy(u_h.at[ti_v], bt_v, s2)
            cf.wait()
            ct.wait()
            pltpu.sync_copy(bf_v, rf_h.at[pl.ds(off, _CH)])
            pltpu.sync_copy(bt_v, rt_h.at[pl.ds(off, _CH)])
            return carry

        lax.fori_loop(0, _NCHUNK, chunk, 0)

    fn = pl.kernel(
        body,
        out_type=(jax.ShapeDtypeStruct((_E, _TBLW), jnp.float32),
                  jax.ShapeDtypeStruct((_E, _TBLW), jnp.float32)),
        mesh=_sc_mesh(),
        scratch_types=[
            pltpu.VMEM((_CH,), jnp.int32),
            pltpu.VMEM((_CH,), jnp.int32),
            pltpu.VMEM((_CH, _TBLW), jnp.float32),
            pltpu.VMEM((_CH, _TBLW), jnp.float32),
            pltpu.SemaphoreType.DMA,
            pltpu.SemaphoreType.DMA,
        ],
    )
    return fn(u, fidx, tidx)


def _sc_scatter(shift, src_pad, dst_pad, zeros_n):
    def body(sh_h, si_h, di_h, z_h, o_h, si_v, di_v, rows_v, acc_s, sem):
        cid = lax.axis_index("c")
        sid = lax.axis_index("s")
        wid = sid * _NC + cid
        rbase = sid * (_NPAD // _NS)
        pltpu.sync_copy(z_h.at[pl.ds(rbase, _NPAD // _NS)],
                        acc_s.at[pl.ds(rbase, _NPAD // _NS)])
        plsc.subcore_barrier()

        def chunk(i, carry):
            off = pl.multiple_of(wid * _SCAP + i * _CH, 8)
            pltpu.sync_copy(si_h.at[pl.ds(off, _CH)], si_v)
            pltpu.sync_copy(di_h.at[pl.ds(off, _CH)], di_v)
            pltpu.async_copy(sh_h.at[si_v], rows_v, sem).wait()
            pltpu.sync_copy(rows_v, acc_s.at[di_v], add=True)
            return carry

        lax.fori_loop(0, _SCH, chunk, 0)
        plsc.subcore_barrier()
        nbase = pl.multiple_of(wid * _NODES_W, 8)
        pltpu.sync_copy(acc_s.at[pl.ds(nbase, _NODES_W)],
                        o_h.at[pl.ds(nbase, _NODES_W)])

    fn = pl.kernel(
        body,
        out_type=jax.ShapeDtypeStruct((_NPAD, _DT), jnp.float32),
        mesh=_sc_mesh(),
        scratch_types=[
            pltpu.VMEM((_CH,), jnp.int32),
            pltpu.VMEM((_CH,), jnp.int32),
            pltpu.VMEM((_CH, _DT), jnp.float32),
            pltpu.VMEM_SHARED((_NPAD, _DT), jnp.float32),
            pltpu.SemaphoreType.DMA,
        ],
    )
    return fn(shift, src_pad, dst_pad, zeros_n)


_BN = 400
_BE = 512


def _tc_updproj(out_prev, sc, Fk, x_s):
    def body(op_ref, sc_ref, fk_ref, xs_ref, out_ref, u_ref):
        out = op_ref[...] + jnp.dot(sc_ref[...], fk_ref[...],
                                    preferred_element_type=jnp.float32)
        out_ref[...] = out
        u_ref[...] = jnp.concatenate([xs_ref[...], out], axis=1)

    return pl.pallas_call(
        body,
        grid=(_N // _BN,),
        in_specs=[
            pl.BlockSpec((_BN, _DT), lambda i: (i, 0)),
            pl.BlockSpec((_BN, _DT), lambda i: (i, 0)),
            pl.BlockSpec((_DT, _DT), lambda i: (0, 0)),
            pl.BlockSpec((_BN, _DS), lambda i: (i, 0)),
        ],
        out_specs=[
            pl.BlockSpec((_BN, _DT), lambda i: (i, 0)),
            pl.BlockSpec((_BN, _TBLW), lambda i: (i, 0)),
        ],
        out_shape=(jax.ShapeDtypeStruct((_N, _DT), jnp.float32),
                   jax.ShapeDtypeStruct((_N, _TBLW), jnp.float32)),
    )(out_prev, sc, Fk, x_s)


def _tc_mlp(rf, rt, ea, W1, b1, a1, W2, b2):
    def body(rf_ref, rt_ref, ea_ref, w1_ref, b1_ref, a1_ref, w2_ref,
             b2_ref, w_ref, dm_ref):
        rf_v = rf_ref[...]
        rt_v = rt_ref[...]
        e_cat = jnp.concatenate(
            [rf_v[:, :_DS], rt_v[:, :_DS], rf_v[:, _DS:], rt_v[:, _DS:],
             ea_ref[...]], axis=1)
        h = jnp.dot(e_cat, w1_ref[...],
                    preferred_element_type=jnp.float32) + b1_ref[...]
        a = a1_ref[0, 0]
        h = jnp.where(h >= 0, h, a * h)
        w = jnp.dot(h, w2_ref[...],
                    preferred_element_type=jnp.float32) + b2_ref[...]
        of = rf_v[:, _DS:]
        ot = rt_v[:, _DS:]
        mf = jnp.sum(of, axis=1, keepdims=True) != 0
        mt = jnp.sum(ot, axis=1, keepdims=True) != 0
        emask = jnp.logical_or(mf, mt).astype(jnp.float32)
        w_ref[...] = w
        dm_ref[...] = (ot - of) * emask

    return pl.pallas_call(
        body,
        grid=(_E // _BE,),
        in_specs=[
            pl.BlockSpec((_BE, _TBLW), lambda i: (i, 0)),
            pl.BlockSpec((_BE, _TBLW), lambda i: (i, 0)),
            pl.BlockSpec((_BE, _DE), lambda i: (i, 0)),
            pl.BlockSpec((_EIN, _HID), lambda i: (0, 0)),
            pl.BlockSpec((1, _HID), lambda i: (0, 0)),
            pl.BlockSpec((1, 1), lambda i: (0, 0)),
            pl.BlockSpec((_HID, _DT), lambda i: (0, 0)),
            pl.BlockSpec((1, _DT), lambda i: (0, 0)),
        ],
        out_specs=[
            pl.BlockSpec((_BE, _DT), lambda i: (i, 0)),
            pl.BlockSpec((_BE, _DT), lambda i: (i, 0)),
        ],
        out_shape=(jax.ShapeDtypeStruct((_E, _DT), jnp.float32),
                   jax.ShapeDtypeStruct((_E, _DT), jnp.float32)),
    )(rf, rt, ea, W1, b1, a1, W2, b2)


def _tc_final(out_prev, sc, Fk):
    def body(op_ref, sc_ref, fk_ref, out_ref):
        out_ref[...] = op_ref[...] + jnp.dot(
            sc_ref[...], fk_ref[...], preferred_element_type=jnp.float32)

    return pl.pallas_call(
        body,
        grid=(_N // _BN,),
        in_specs=[
            pl.BlockSpec((_BN, _DT), lambda i: (i, 0)),
            pl.BlockSpec((_BN, _DT), lambda i: (i, 0)),
            pl.BlockSpec((_DT, _DT), lambda i: (0, 0)),
        ],
        out_specs=pl.BlockSpec((_BN, _DT), lambda i: (i, 0)),
        out_shape=jax.ShapeDtypeStruct((_N, _DT), jnp.float32),
    )(out_prev, sc, Fk)


def kernel(x_s, x_t, edge_index, edge_attr, W1, b1, a1, W2, b2, F):
    fidx = edge_index[0]
    tidx = edge_index[1]
    b1r = b1.reshape(1, _HID)
    b2r = b2.reshape(1, _DT)
    a1r = a1.reshape(1, 1)
    zeros_n = jnp.zeros((_NPAD, _DT), jnp.float32)

    perm = jnp.argsort(tidx, stable=True)
    ti_s = tidx[perm].astype(jnp.int32)
    fi_p = fidx[perm].astype(jnp.int32)
    ea_p = edge_attr[perm]
    r0 = jnp.searchsorted(ti_s, jnp.arange(_NW, dtype=jnp.int32) * _NODES_W
                          ).astype(jnp.int32)
    r1 = jnp.searchsorted(ti_s, (jnp.arange(_NW, dtype=jnp.int32) + 1)
                          * _NODES_W).astype(jnp.int32)
    pos = r0[:, None] + jnp.arange(_SCAP, dtype=jnp.int32)[None, :]
    valid = pos < r1[:, None]
    src_pad = jnp.where(valid, jnp.minimum(pos, _E - 1), 0
                        ).astype(jnp.int32).reshape(-1)
    dst_pad = jnp.where(valid, ti_s[jnp.minimum(pos, _E - 1)], _DUMP
                        ).astype(jnp.int32).reshape(-1)

    out = jnp.zeros((_N, _DT), jnp.float32)
    sc = x_t
    for k in range(_K):
        out, u = _tc_updproj(out, sc, F[k], x_s)
        rf, rt = _sc_gather(u, fi_p, ti_s)
        w, dm = _tc_mlp(rf, rt, ea_p, W1, b1r, a1r, W2, b2r)
        nrm = jnp.sqrt(jnp.sum(w * w, axis=1, keepdims=True))
        wn = w / nrm
        wn = jnp.where(jnp.isnan(wn), 0.0, wn)
        shift = dm * wn
        sc = _sc_scatter(shift, src_pad, dst_pad, zeros_n)[:_N]
    return _tc_final(out, sc, F[_K])
